# Initial kernel scaffold; baseline (speedup 1.0000x reference)
import jax
import jax.numpy as jnp
from jax.experimental import pallas as pl


def kernel(inputs, token_emb, pos_emb):
    raise NotImplementedError("write your pallas kernel here")


# speedup vs baseline: 2.5956x; 2.5956x over previous
import functools

import jax
import jax.numpy as jnp
from jax import lax
from jax.experimental import pallas as pl
from jax.experimental.pallas import tpu as pltpu
from jax.experimental.pallas import tpu_sc as plsc

MAXLEN = 200
EMBED = 64
BATCH = 1024

NC = 2
NS = 16
NW = NC * NS
ROWS_PER_WORKER = BATCH // NW
LANES = 16


def _body(idx_hbm, table_hbm, pos_hbm, out_hbm, idx_v, rows_v, pos_v, sem):
    wid = lax.axis_index("s") * NC + lax.axis_index("c")

    pltpu.sync_copy(pos_hbm, pos_v)

    def row_body(r, carry):
        base = (wid * ROWS_PER_WORKER + r) * MAXLEN
        pltpu.sync_copy(idx_hbm.at[pl.ds(base, MAXLEN)], idx_v)
        pltpu.async_copy(table_hbm.at[idx_v], rows_v, sem).wait()

        def add_body(t, c2):
            for c in range(EMBED // LANES):
                sl = pl.ds(c * LANES, LANES)
                plsc.addupdate(rows_v.at[t, sl], pos_v[t, sl])
            return c2

        lax.fori_loop(0, MAXLEN, add_body, 0, unroll=2)
        pltpu.sync_copy(rows_v, out_hbm.at[pl.ds(base, MAXLEN)])
        return carry

    lax.fori_loop(0, ROWS_PER_WORKER, row_body, 0)


@jax.jit
def kernel(inputs, token_emb, pos_emb):
    n = BATCH * MAXLEN
    idx_flat = inputs.reshape(n).astype(jnp.int32)
    mesh = plsc.VectorSubcoreMesh(core_axis_name="c", subcore_axis_name="s")
    run = functools.partial(
        pl.kernel,
        out_type=jax.ShapeDtypeStruct((n, EMBED), jnp.float32),
        mesh=mesh,
        scratch_types=[
            pltpu.VMEM((MAXLEN,), jnp.int32),
            pltpu.VMEM((MAXLEN, EMBED), jnp.float32),
            pltpu.VMEM((MAXLEN, EMBED), jnp.float32),
            pltpu.SemaphoreType.DMA,
        ],
        compiler_params=pltpu.CompilerParams(use_tc_tiling_on_sc=False),
    )(_body)
    out = run(idx_flat, token_emb, pos_emb)
    return out.reshape(BATCH, MAXLEN, EMBED)


# speedup vs baseline: 3.1903x; 1.2291x over previous
import functools

import jax
import jax.numpy as jnp
from jax import lax
from jax.experimental import pallas as pl
from jax.experimental.pallas import tpu as pltpu
from jax.experimental.pallas import tpu_sc as plsc

MAXLEN = 200
EMBED = 64
BATCH = 1024

NC = 2
NS = 16
NW = NC * NS
LANES = 16

CHUNK = 800
REP = CHUNK // MAXLEN
PER_WORKER = BATCH * MAXLEN // NW
NCHUNK = PER_WORKER // CHUNK
NPAIR = NCHUNK // 2


def _body(idx_hbm, table_hbm, pos_hbm, out_hbm,
          idx0, idx1, rows0, rows1, pos_v, g0, g1, o0, o1):
    wid = lax.axis_index("s") * NC + lax.axis_index("c")
    wbase = wid * PER_WORKER

    idx = (idx0, idx1)
    rows = (rows0, rows1)
    gsem = (g0, g1)
    osem = (o0, o1)

    pltpu.sync_copy(pos_hbm, pos_v)

    def fetch(c, b):
        base = wbase + c * CHUNK
        pltpu.sync_copy(idx_hbm.at[pl.ds(base, CHUNK)], idx[b])
        pltpu.async_copy(table_hbm.at[idx[b]], rows[b], gsem[b])

    def wait_gather(b):
        pltpu.make_async_copy(table_hbm.at[idx[b]], rows[b], gsem[b]).wait()

    def flush(c, b):
        base = wbase + c * CHUNK
        pltpu.async_copy(rows[b], out_hbm.at[pl.ds(base, CHUNK)], osem[b])

    def wait_out(b):
        pltpu.make_async_copy(
            rows[b], out_hbm.at[pl.ds(wbase, CHUNK)], osem[b]).wait()

    def add_pos(b):
        rv = rows[b]

        def t_body(t, carry):
            for c in range(EMBED // LANES):
                sl = pl.ds(c * LANES, LANES)
                p = pos_v[t, sl]
                for j in range(REP):
                    plsc.addupdate(rv.at[t + j * MAXLEN, sl], p)
            return carry

        lax.fori_loop(0, MAXLEN, t_body, 0, unroll=2)

    fetch(0, 0)

    def pair_body(k, carry):
        c0 = 2 * k
        c1 = 2 * k + 1

        @pl.when(k > 0)
        def _():
            wait_out(1)

        fetch(c1, 1)
        wait_gather(0)
        add_pos(0)
        flush(c0, 0)
        wait_gather(1)
        add_pos(1)
        wait_out(0)

        @pl.when(k < NPAIR - 1)
        def _():
            fetch(c0 + 2, 0)

        flush(c1, 1)
        return carry

    lax.fori_loop(0, NPAIR, pair_body, 0)
    wait_out(1)


@jax.jit
def kernel(inputs, token_emb, pos_emb):
    n = BATCH * MAXLEN
    idx_flat = inputs.reshape(n).astype(jnp.int32)
    mesh = plsc.VectorSubcoreMesh(core_axis_name="c", subcore_axis_name="s")
    run = functools.partial(
        pl.kernel,
        out_type=jax.ShapeDtypeStruct((n, EMBED), jnp.float32),
        mesh=mesh,
        scratch_types=[
            pltpu.VMEM((CHUNK,), jnp.int32),
            pltpu.VMEM((CHUNK,), jnp.int32),
            pltpu.VMEM((CHUNK, EMBED), jnp.float32),
            pltpu.VMEM((CHUNK, EMBED), jnp.float32),
            pltpu.VMEM((MAXLEN, EMBED), jnp.float32),
            pltpu.SemaphoreType.DMA,
            pltpu.SemaphoreType.DMA,
            pltpu.SemaphoreType.DMA,
            pltpu.SemaphoreType.DMA,
        ],
        compiler_params=pltpu.CompilerParams(use_tc_tiling_on_sc=False),
    )(_body)
    out = run(idx_flat, token_emb, pos_emb)
    return out.reshape(BATCH, MAXLEN, EMBED)
